# baseline (speedup 1.0000x reference)
import functools

import jax
import jax.numpy as jnp
from jax import lax
from jax.experimental import pallas as pl
from jax.experimental.pallas import tpu as pltpu
from jax.experimental.pallas import tpu_sc as plsc

_NW = 32
_P = 784
_BP = 512


def _idx_body(a_ref, bnd_ref, str_ref, off_ref, c_ref, idx_ref):
    vox_f = jnp.round(jnp.dot(c_ref[...], a_ref[...]))
    vox = jnp.clip(vox_f, 0.0, bnd_ref[...]).astype(jnp.int32)
    base = jnp.sum(vox * str_ref[...], axis=1, keepdims=True)
    idx_ref[...] = base + off_ref[...]


def _softmax_body(g_ref, out_ref):
    x = g_ref[...]
    m = jnp.max(x, axis=1, keepdims=True)
    e = jnp.exp(x - m)
    y = e / jnp.sum(e, axis=1, keepdims=True)
    out_ref[...] = jnp.concatenate(
        [y, jnp.zeros((y.shape[0], 1), y.dtype)], axis=1)


def _make_sc_gather(B, C, b_per_w):
    n_chunks = b_per_w // _P
    chunk = _P * C
    mesh = plsc.VectorSubcoreMesh(core_axis_name="c", subcore_axis_name="s")

    @functools.partial(
        pl.kernel,
        mesh=mesh,
        compiler_params=pltpu.CompilerParams(use_tc_tiling_on_sc=False),
        out_type=jax.ShapeDtypeStruct((B * C,), jnp.float32),
        scratch_types=[
            pltpu.VMEM((chunk,), jnp.int32),
            pltpu.VMEM((chunk,), jnp.float32),
            pltpu.SemaphoreType.DMA,
        ],
    )
    def gk(idx_hbm, table_hbm, out_hbm, idx_v, g_v, sem):
        wid = lax.axis_index("s") * 2 + lax.axis_index("c")
        for k in range(n_chunks):
            off = (wid * n_chunks + k) * chunk
            pltpu.sync_copy(idx_hbm.at[pl.ds(off, chunk)], idx_v)
            pltpu.async_copy(table_hbm.at[idx_v], g_v, sem).wait()
            pltpu.sync_copy(g_v, out_hbm.at[pl.ds(off, chunk)])

    return gk


def kernel(streamlines, padding_mask, step, fodf, affine, inverse_affine):
    N = streamlines.shape[0]
    xdim, ydim, zdim, C = fodf.shape

    per_blk = _NW * _P
    b_per_w = -(-N // per_blk) * _P
    B = b_per_w * _NW

    coords = lax.dynamic_index_in_dim(streamlines, step, axis=1,
                                      keepdims=False)
    ones = jnp.ones((N, 1), streamlines.dtype)
    coords_h = jnp.pad(jnp.concatenate([coords, ones], axis=1),
                       ((0, B - N), (0, 0)))

    amat = inverse_affine[:3, :].T
    bnd = jnp.array([[xdim - 1, ydim - 1, zdim - 1]], jnp.float32)
    strides = jnp.array([[C * ydim * zdim, zdim, 1]], jnp.int32)
    offs = (jnp.arange(C, dtype=jnp.int32) * (ydim * zdim))[None, :]

    idx = pl.pallas_call(
        _idx_body,
        grid=(B // _BP,),
        out_shape=jax.ShapeDtypeStruct((B, C), jnp.int32),
        in_specs=[
            pl.BlockSpec((4, 3), lambda i: (0, 0)),
            pl.BlockSpec((1, 3), lambda i: (0, 0)),
            pl.BlockSpec((1, 3), lambda i: (0, 0)),
            pl.BlockSpec((1, C), lambda i: (0, 0)),
            pl.BlockSpec((_BP, 4), lambda i: (i, 0)),
        ],
        out_specs=pl.BlockSpec((_BP, C), lambda i: (i, 0)),
    )(amat, bnd, strides, offs, coords_h)

    table = fodf.transpose(0, 3, 1, 2).reshape(-1)
    gathered = _make_sc_gather(B, C, b_per_w)(idx.reshape(-1), table)

    out = pl.pallas_call(
        _softmax_body,
        grid=(B // _BP,),
        out_shape=jax.ShapeDtypeStruct((B, C + 1), jnp.float32),
        in_specs=[pl.BlockSpec((_BP, C), lambda i: (i, 0))],
        out_specs=pl.BlockSpec((_BP, C + 1), lambda i: (i, 0)),
    )(gathered.reshape(B, C))

    return out[:N]
